# Initial kernel scaffold; baseline (speedup 1.0000x reference)
import jax
import jax.numpy as jnp
from jax.experimental import pallas as pl


def kernel(basic_block, edge_index, W1, b1, W2, b2):
    raise NotImplementedError("write your pallas kernel here")


# speedup vs baseline: 16.2019x; 16.2019x over previous
import functools
import math

import jax
import jax.numpy as jnp
from jax import lax
from jax.experimental import pallas as pl
from jax.experimental.pallas import tpu as pltpu
from jax.experimental.pallas import tpu_sc as plsc

N = 10000
D = 128
E = 320000
NP = 10240
BM = 1024
CHUNK = 128
NCHUNK = E // CHUNK
NW = 32
FULL_J = NCHUNK // NW
EXTRA = NCHUNK - FULL_J * NW
RPW = NP // 16

def _wid_nj():
    core = lax.axis_index("c")
    sub = lax.axis_index("s")
    wid = sub * 2 + core
    nj = FULL_J + jnp.where(wid < EXTRA, 1, 0).astype(jnp.int32)
    return core, sub, wid, nj


def _deg_body(dst_hbm, dega_hbm, degb_hbm, idx_v, ones_v, zbuf_v, deg_sh):
    core, sub, wid, nj = _wid_nj()

    @pl.loop(0, CHUNK // 16)
    def _fill_ones(i):
        ones_v[pl.ds(i * 16, 16)] = jnp.full((16,), 1.0, jnp.float32)

    @pl.loop(0, RPW // 16)
    def _fill_zero(i):
        zbuf_v[pl.ds(i * 16, 16)] = jnp.zeros((16,), jnp.float32)

    pltpu.sync_copy(zbuf_v, deg_sh.at[pl.ds(pl.multiple_of(sub * RPW, RPW), RPW)])
    plsc.subcore_barrier()

    @pl.loop(0, nj)
    def _count(j):
        base = pl.multiple_of((j * NW + wid) * CHUNK, CHUNK)
        pltpu.sync_copy(dst_hbm.at[pl.ds(base, CHUNK)], idx_v)
        pltpu.sync_copy(ones_v, deg_sh.at[idx_v], add=True)

    plsc.subcore_barrier()
    off = pl.multiple_of(sub * RPW, RPW)

    @pl.when(core == 0)
    def _():
        pltpu.sync_copy(deg_sh.at[pl.ds(off, RPW)], dega_hbm.at[pl.ds(off, RPW)])

    @pl.when(core == 1)
    def _():
        pltpu.sync_copy(deg_sh.at[pl.ds(off, RPW)], degb_hbm.at[pl.ds(off, RPW)])


def _scatter_body(h_hbm, src_hbm, dst_hbm, pa_hbm, pb_hbm,
                  sidx_v, didx_v, rows_v, acc_sh, sem):
    core, sub, wid, nj = _wid_nj()

    @pl.loop(0, CHUNK)
    def _zrow(r):
        for cc in range(D // 16):
            rows_v[r, pl.ds(cc * 16, 16)] = jnp.zeros((16,), jnp.float32)

    for k in range(RPW // CHUNK):
        off = pl.multiple_of(sub * RPW + k * CHUNK, CHUNK)
        pltpu.sync_copy(rows_v, acc_sh.at[pl.ds(off, CHUNK)])
    plsc.subcore_barrier()

    @pl.loop(0, nj)
    def _edges(j):
        base = pl.multiple_of((j * NW + wid) * CHUNK, CHUNK)
        pltpu.sync_copy(src_hbm.at[pl.ds(base, CHUNK)], sidx_v)
        pltpu.sync_copy(dst_hbm.at[pl.ds(base, CHUNK)], didx_v)
        pltpu.async_copy(h_hbm.at[sidx_v], rows_v, sem).wait()
        pltpu.sync_copy(rows_v, acc_sh.at[didx_v], add=True)

    plsc.subcore_barrier()

    @pl.when(core == 0)
    def _():
        for k in range(RPW // CHUNK):
            off = pl.multiple_of(sub * RPW + k * CHUNK, CHUNK)
            pltpu.sync_copy(acc_sh.at[pl.ds(off, CHUNK)], pa_hbm.at[pl.ds(off, CHUNK)])

    @pl.when(core == 1)
    def _():
        for k in range(RPW // CHUNK):
            off = pl.multiple_of(sub * RPW + k * CHUNK, CHUNK)
            pltpu.sync_copy(acc_sh.at[pl.ds(off, CHUNK)], pb_hbm.at[pl.ds(off, CHUNK)])


@functools.lru_cache(maxsize=None)
def _sc_kernels():
    mesh = plsc.VectorSubcoreMesh(
        core_axis_name="c", subcore_axis_name="s", num_cores=2, num_subcores=16
    )
    deg = pl.kernel(
        _deg_body,
        out_type=(
            jax.ShapeDtypeStruct((NP,), jnp.float32),
            jax.ShapeDtypeStruct((NP,), jnp.float32),
        ),
        mesh=mesh,
        scratch_types=[
            pltpu.VMEM((CHUNK,), jnp.int32),
            pltpu.VMEM((CHUNK,), jnp.float32),
            pltpu.VMEM((RPW,), jnp.float32),
            pltpu.VMEM_SHARED((NP,), jnp.float32),
        ],
    )
    scatter = pl.kernel(
        _scatter_body,
        out_type=(
            jax.ShapeDtypeStruct((NP, D), jnp.float32),
            jax.ShapeDtypeStruct((NP, D), jnp.float32),
        ),
        mesh=mesh,
        scratch_types=[
            pltpu.VMEM((CHUNK,), jnp.int32),
            pltpu.VMEM((CHUNK,), jnp.int32),
            pltpu.VMEM((CHUNK, D), jnp.float32),
            pltpu.VMEM_SHARED((NP, D), jnp.float32),
            pltpu.SemaphoreType.DMA,
        ],
    )
    return deg, scatter


def _t1_body(x_ref, w_ref, dega_ref, degb_ref, h_ref, dis_ref):
    deg = dega_ref[...] + degb_ref[...] + 1.0
    dis = lax.rsqrt(deg)
    dis_ref[...] = dis
    h_ref[...] = (
        jnp.dot(x_ref[...], w_ref[...], preferred_element_type=jnp.float32) * dis
    )


_t1 = pl.pallas_call(
    _t1_body,
    grid=(NP // BM,),
    in_specs=[
        pl.BlockSpec((BM, D), lambda i: (i, 0)),
        pl.BlockSpec((D, D), lambda i: (0, 0)),
        pl.BlockSpec((BM, 1), lambda i: (i, 0)),
        pl.BlockSpec((BM, 1), lambda i: (i, 0)),
    ],
    out_specs=[
        pl.BlockSpec((BM, D), lambda i: (i, 0)),
        pl.BlockSpec((BM, 1), lambda i: (i, 0)),
    ],
    out_shape=[
        jax.ShapeDtypeStruct((NP, D), jnp.float32),
        jax.ShapeDtypeStruct((NP, 1), jnp.float32),
    ],
)

_NEG_LOG_OVER_D = -math.log(10000.0) / D


def _t2_body(pa_ref, pb_ref, h1_ref, dis_ref, b1_ref, w2_ref, h2_ref):
    i = pl.program_id(0)
    dis = dis_ref[...]
    agg = dis * (pa_ref[...] + pb_ref[...] + h1_ref[...]) + b1_ref[...]
    pos = (lax.broadcasted_iota(jnp.int32, (BM, D), 0) + i * BM).astype(jnp.float32)
    col = lax.broadcasted_iota(jnp.int32, (BM, D), 1)
    even_exp = ((col // 2) * 2).astype(jnp.float32)
    ang = pos * jnp.exp(even_exp * _NEG_LOG_OVER_D)
    pe = jnp.where(col % 2 == 0, jnp.sin(ang), jnp.cos(ang))
    x1 = jnp.maximum(agg + pe, 0.0)
    h2_ref[...] = (
        jnp.dot(x1, w2_ref[...], preferred_element_type=jnp.float32) * dis
    )


_t2 = pl.pallas_call(
    _t2_body,
    grid=(NP // BM,),
    in_specs=[
        pl.BlockSpec((BM, D), lambda i: (i, 0)),
        pl.BlockSpec((BM, D), lambda i: (i, 0)),
        pl.BlockSpec((BM, D), lambda i: (i, 0)),
        pl.BlockSpec((BM, 1), lambda i: (i, 0)),
        pl.BlockSpec((1, D), lambda i: (0, 0)),
        pl.BlockSpec((D, D), lambda i: (0, 0)),
    ],
    out_specs=pl.BlockSpec((BM, D), lambda i: (i, 0)),
    out_shape=jax.ShapeDtypeStruct((NP, D), jnp.float32),
)


def _t3_body(qa_ref, qb_ref, h2_ref, dis_ref, b2_ref, out_ref):
    out_ref[...] = (
        dis_ref[...] * (qa_ref[...] + qb_ref[...] + h2_ref[...]) + b2_ref[...]
    )


_t3 = pl.pallas_call(
    _t3_body,
    grid=(NP // BM,),
    in_specs=[
        pl.BlockSpec((BM, D), lambda i: (i, 0)),
        pl.BlockSpec((BM, D), lambda i: (i, 0)),
        pl.BlockSpec((BM, D), lambda i: (i, 0)),
        pl.BlockSpec((BM, 1), lambda i: (i, 0)),
        pl.BlockSpec((1, D), lambda i: (0, 0)),
    ],
    out_specs=pl.BlockSpec((BM, D), lambda i: (i, 0)),
    out_shape=jax.ShapeDtypeStruct((NP, D), jnp.float32),
)


def kernel(basic_block, edge_index, W1, b1, W2, b2):
    ei = edge_index.astype(jnp.int32)
    src = ei[0]
    dst = ei[1]
    x = jnp.pad(basic_block, ((0, NP - N), (0, 0)))

    _deg_kernel, _scatter_kernel = _sc_kernels()
    dega, degb = _deg_kernel(dst)
    h1p, dis = _t1(x, W1, dega.reshape(NP, 1), degb.reshape(NP, 1))
    pa, pb = _scatter_kernel(h1p, src, dst)
    h2p = _t2(pa, pb, h1p, dis, b1.reshape(1, D), W2)
    qa, qb = _scatter_kernel(h2p, src, dst)
    out = _t3(qa, qb, h2p, dis, b2.reshape(1, D))
    return out[:N]
